# baseline (speedup 1.0000x reference)
import functools

import jax
import jax.numpy as jnp
from jax.experimental import pallas as pl
from jax.experimental.pallas import tpu as pltpu

_PREC = jax.lax.Precision.HIGHEST
_LANES = 128


def _router_body(e_num, s_len, alpha_ref, x_ref, wr_ref, idx_ref, gate_ref):
    pooled = jnp.sum(x_ref[0], axis=0, keepdims=True) * (1.0 / s_len)
    logits = jnp.dot(pooled, wr_ref[...], precision=_PREC,
                     preferred_element_type=jnp.float32)
    lane = jax.lax.broadcasted_iota(jnp.int32, logits.shape, 1)
    valid = lane < e_num
    l = jnp.where(valid, logits, jnp.float32(-1e30))
    m = jnp.max(l)
    p = jnp.exp(l - m)
    p = jnp.where(valid, p, 0.0)
    p = p / jnp.sum(p)
    big = jnp.int32(1 << 20)
    v1 = jnp.max(p)
    i1 = jnp.min(jnp.where(p >= v1, lane, big))
    p2 = jnp.where(lane == i1, jnp.float32(-1.0), p)
    v2 = jnp.max(p2)
    i2 = jnp.min(jnp.where(p2 >= v2, lane, big))
    a = jnp.clip(alpha_ref[0], 0.1, 1.0)
    scale = (1.0 - a) / (v1 + v2)
    g1 = scale * v1
    g2 = scale * v2
    gate_row = jnp.where(lane == 0, a,
                         jnp.where(lane == 1, g1,
                                   jnp.where(lane == 2, g2, 0.0)))
    idx_row = jnp.where(lane == 1, i1 + 1, jnp.where(lane == 2, i2 + 1, 0))
    gate_ref[0] = gate_row
    idx_ref[0] = idx_row.astype(jnp.int32)


def _moe_body(bidx_ref, gate_ref, x_ref, w1_ref, b1_ref, w2_ref, b2_ref,
              out_ref):
    b = pl.program_id(0)
    fb = pl.program_id(1)
    br = pl.program_id(2)
    g = gate_ref[b, br]
    h = jnp.dot(x_ref[0], w1_ref[0], precision=_PREC,
                preferred_element_type=jnp.float32)
    h = jax.nn.gelu(h + b1_ref[0, 0])
    y = jnp.dot(h, w2_ref[0], precision=_PREC,
                preferred_element_type=jnp.float32)
    y = y * g

    first = jnp.logical_and(fb == 0, br == 0)

    @pl.when(first)
    def _init():
        out_ref[0] = y + g * b2_ref[0, 0]

    @pl.when(jnp.logical_and(fb == 0, br != 0))
    def _acc_with_bias():
        out_ref[0] += y + g * b2_ref[0, 0]

    @pl.when(fb != 0)
    def _acc():
        out_ref[0] += y


def kernel(x, W1, b1, W2, b2, Wr, eW1, eb1, eW2, eb2, alpha):
    B, S, D = x.shape
    F = W1.shape[1]
    E = eW1.shape[0]
    NBR = 3

    wr_pad = jnp.zeros((D, _LANES), jnp.float32).at[:, :E].set(Wr)
    alpha_arr = jnp.reshape(alpha.astype(jnp.float32), (1,))
    idx_pad, gate_pad = pl.pallas_call(
        functools.partial(_router_body, E, S),
        grid=(B,),
        in_specs=[
            pl.BlockSpec(memory_space=pltpu.SMEM),
            pl.BlockSpec((1, S, D), lambda b: (b, 0, 0)),
            pl.BlockSpec((D, _LANES), lambda b: (0, 0)),
        ],
        out_specs=[
            pl.BlockSpec((1, 1, _LANES), lambda b: (b, 0, 0)),
            pl.BlockSpec((1, 1, _LANES), lambda b: (b, 0, 0)),
        ],
        out_shape=[
            jax.ShapeDtypeStruct((B, 1, _LANES), jnp.int32),
            jax.ShapeDtypeStruct((B, 1, _LANES), jnp.float32),
        ],
    )(alpha_arr, x, wr_pad)
    bidx = idx_pad[:, 0, :NBR]
    gates = gate_pad[:, 0, :NBR]

    aW1 = jnp.concatenate([W1[None], eW1], axis=0)
    ab1 = jnp.concatenate([b1[None], eb1], axis=0)[:, None, :]
    aW2 = jnp.concatenate([W2[None], eW2], axis=0)
    ab2 = jnp.concatenate([b2[None], eb2], axis=0)[:, None, :]

    FBLK = 512 if F % 512 == 0 else F
    NFB = F // FBLK

    grid_spec = pltpu.PrefetchScalarGridSpec(
        num_scalar_prefetch=2,
        grid=(B, NFB, NBR),
        in_specs=[
            pl.BlockSpec((1, S, D), lambda b, fb, br, bidx, gates: (b, 0, 0)),
            pl.BlockSpec((1, D, FBLK),
                         lambda b, fb, br, bidx, gates: (bidx[b, br], 0, fb)),
            pl.BlockSpec((1, 1, FBLK),
                         lambda b, fb, br, bidx, gates: (bidx[b, br], 0, fb)),
            pl.BlockSpec((1, FBLK, D),
                         lambda b, fb, br, bidx, gates: (bidx[b, br], fb, 0)),
            pl.BlockSpec((1, 1, D),
                         lambda b, fb, br, bidx, gates: (bidx[b, br], 0, 0)),
        ],
        out_specs=pl.BlockSpec((1, S, D),
                               lambda b, fb, br, bidx, gates: (b, 0, 0)),
    )
    out = pl.pallas_call(
        _moe_body,
        grid_spec=grid_spec,
        out_shape=jax.ShapeDtypeStruct((B, S, D), jnp.float32),
        compiler_params=pltpu.CompilerParams(
            dimension_semantics=("arbitrary", "arbitrary", "arbitrary"),
            vmem_limit_bytes=100 * 1024 * 1024,
        ),
    )(bidx, gates, x, aW1, ab1, aW2, ab2)
    return out


# speedup vs baseline: 4.1438x; 4.1438x over previous
import functools

import jax
import jax.numpy as jnp
from jax.experimental import pallas as pl
from jax.experimental.pallas import tpu as pltpu

_PREC = jax.lax.Precision.DEFAULT
_LANES = 128


def _router_body(e_num, s_len, alpha_ref, x_ref, wr_ref, idx_ref, gate_ref):
    pooled = jnp.sum(x_ref[0], axis=0, keepdims=True) * (1.0 / s_len)
    logits = jnp.dot(pooled, wr_ref[...], precision=_PREC,
                     preferred_element_type=jnp.float32)
    lane = jax.lax.broadcasted_iota(jnp.int32, logits.shape, 1)
    valid = lane < e_num
    l = jnp.where(valid, logits, jnp.float32(-1e30))
    m = jnp.max(l)
    p = jnp.exp(l - m)
    p = jnp.where(valid, p, 0.0)
    p = p / jnp.sum(p)
    big = jnp.int32(1 << 20)
    v1 = jnp.max(p)
    i1 = jnp.min(jnp.where(p >= v1, lane, big))
    p2 = jnp.where(lane == i1, jnp.float32(-1.0), p)
    v2 = jnp.max(p2)
    i2 = jnp.min(jnp.where(p2 >= v2, lane, big))
    a = jnp.clip(alpha_ref[0], 0.1, 1.0)
    scale = (1.0 - a) / (v1 + v2)
    g1 = scale * v1
    g2 = scale * v2
    gate_row = jnp.where(lane == 0, a,
                         jnp.where(lane == 1, g1,
                                   jnp.where(lane == 2, g2, 0.0)))
    idx_row = jnp.where(lane == 1, i1 + 1, jnp.where(lane == 2, i2 + 1, 0))
    gate_ref[0] = gate_row
    idx_ref[0] = idx_row.astype(jnp.int32)


def _moe_body(bidx_ref, gate_ref, x_ref, w1_ref, b1_ref, w2_ref, b2_ref,
              out_ref):
    b = pl.program_id(0)
    fb = pl.program_id(1)
    br = pl.program_id(2)
    g = gate_ref[b, br]
    h = jnp.dot(x_ref[0], w1_ref[0], precision=_PREC,
                preferred_element_type=jnp.float32)
    h = jax.nn.gelu(h + b1_ref[0, 0])
    y = jnp.dot(h, w2_ref[0], precision=_PREC,
                preferred_element_type=jnp.float32)
    y = y * g

    first = jnp.logical_and(fb == 0, br == 0)

    @pl.when(first)
    def _init():
        out_ref[0] = y + g * b2_ref[0, 0]

    @pl.when(jnp.logical_and(fb == 0, br != 0))
    def _acc_with_bias():
        out_ref[0] += y + g * b2_ref[0, 0]

    @pl.when(fb != 0)
    def _acc():
        out_ref[0] += y


def kernel(x, W1, b1, W2, b2, Wr, eW1, eb1, eW2, eb2, alpha):
    B, S, D = x.shape
    F = W1.shape[1]
    E = eW1.shape[0]
    NBR = 3

    wr_pad = jnp.zeros((D, _LANES), jnp.float32).at[:, :E].set(Wr)
    alpha_arr = jnp.reshape(alpha.astype(jnp.float32), (1,))
    idx_pad, gate_pad = pl.pallas_call(
        functools.partial(_router_body, E, S),
        grid=(B,),
        in_specs=[
            pl.BlockSpec(memory_space=pltpu.SMEM),
            pl.BlockSpec((1, S, D), lambda b: (b, 0, 0)),
            pl.BlockSpec((D, _LANES), lambda b: (0, 0)),
        ],
        out_specs=[
            pl.BlockSpec((1, 1, _LANES), lambda b: (b, 0, 0)),
            pl.BlockSpec((1, 1, _LANES), lambda b: (b, 0, 0)),
        ],
        out_shape=[
            jax.ShapeDtypeStruct((B, 1, _LANES), jnp.int32),
            jax.ShapeDtypeStruct((B, 1, _LANES), jnp.float32),
        ],
    )(alpha_arr, x, wr_pad)
    bidx = idx_pad[:, 0, :NBR]
    gates = gate_pad[:, 0, :NBR]

    aW1 = jnp.concatenate([W1[None], eW1], axis=0)
    ab1 = jnp.concatenate([b1[None], eb1], axis=0)[:, None, :]
    aW2 = jnp.concatenate([W2[None], eW2], axis=0)
    ab2 = jnp.concatenate([b2[None], eb2], axis=0)[:, None, :]

    FBLK = 512 if F % 512 == 0 else F
    NFB = F // FBLK

    grid_spec = pltpu.PrefetchScalarGridSpec(
        num_scalar_prefetch=2,
        grid=(B, NFB, NBR),
        in_specs=[
            pl.BlockSpec((1, S, D), lambda b, fb, br, bidx, gates: (b, 0, 0)),
            pl.BlockSpec((1, D, FBLK),
                         lambda b, fb, br, bidx, gates: (bidx[b, br], 0, fb)),
            pl.BlockSpec((1, 1, FBLK),
                         lambda b, fb, br, bidx, gates: (bidx[b, br], 0, fb)),
            pl.BlockSpec((1, FBLK, D),
                         lambda b, fb, br, bidx, gates: (bidx[b, br], fb, 0)),
            pl.BlockSpec((1, 1, D),
                         lambda b, fb, br, bidx, gates: (bidx[b, br], 0, 0)),
        ],
        out_specs=pl.BlockSpec((1, S, D),
                               lambda b, fb, br, bidx, gates: (b, 0, 0)),
    )
    out = pl.pallas_call(
        _moe_body,
        grid_spec=grid_spec,
        out_shape=jax.ShapeDtypeStruct((B, S, D), jnp.float32),
        compiler_params=pltpu.CompilerParams(
            dimension_semantics=("arbitrary", "arbitrary", "arbitrary"),
            vmem_limit_bytes=100 * 1024 * 1024,
        ),
    )(bidx, gates, x, aW1, ab1, aW2, ab2)
    return out
